# baseline (speedup 1.0000x reference)
import functools

import jax
import jax.numpy as jnp
from jax import lax
from jax.experimental import pallas as pl
from jax.experimental.pallas import tpu as pltpu
from jax.experimental.pallas import tpu_sc as plsc

B = 128
V = 100000
NC = 2
NS = 16
L = 16
NW = NC * NS
RPW = B // NW
NCHUNK = V // L
UNROLL = 10

_LN2 = 0.6931471805599453
_SQRT2 = 1.4142135623730951


def _ln(x):
    bits = plsc.bitcast(x, jnp.int32)
    e = lax.shift_right_arithmetic(bits, 23) - 127
    mbits = lax.bitwise_or(lax.bitwise_and(bits, 0x7FFFFF), 0x3F800000)
    m = plsc.bitcast(mbits, jnp.float32)
    big = m > _SQRT2
    m = jnp.where(big, m * 0.5, m)
    e = jnp.where(big, e + 1, e)
    t = (m - 1.0) / (m + 1.0)
    t2 = t * t
    p = jnp.float32(1.0 / 9.0)
    p = p * t2 + jnp.float32(1.0 / 7.0)
    p = p * t2 + jnp.float32(1.0 / 5.0)
    p = p * t2 + jnp.float32(1.0 / 3.0)
    p = p * t2 + 1.0
    return e.astype(jnp.float32) * _LN2 + 2.0 * t * p


_mesh = plsc.VectorSubcoreMesh(
    core_axis_name="c", subcore_axis_name="s", num_cores=NC, num_subcores=NS
)


@functools.partial(
    pl.kernel,
    out_type=(
        jax.ShapeDtypeStruct((NW, L), jnp.float32),
        jax.ShapeDtypeStruct((NW, L), jnp.int32),
    ),
    mesh=_mesh,
    compiler_params=pltpu.CompilerParams(needs_layout_passes=False),
    scratch_types=[
        pltpu.VMEM((V,), jnp.float32),
        pltpu.VMEM((B,), jnp.int32),
        pltpu.VMEM((L,), jnp.float32),
        pltpu.VMEM((L,), jnp.int32),
    ],
)
def _sc_kern(logits_hbm, act_hbm, lp_hbm, mode_hbm, row_v, act_v, lp_v, mode_v):
    cid = lax.axis_index("c")
    sid = lax.axis_index("s")
    wid = sid * NC + cid
    pltpu.sync_copy(act_hbm, act_v)
    lanes = lax.iota(jnp.int32, L)

    lp_acc = jnp.zeros((L,), jnp.float32)
    mode_acc = jnp.zeros((L,), jnp.int32)

    for j in range(RPW):
        r = wid * RPW + j
        pltpu.sync_copy(logits_hbm.at[r], row_v)

        def pass_max(k, carry):
            m, mi = carry
            base = k * (L * UNROLL)
            for u in range(UNROLL):
                v = row_v[pl.ds(base + u * L, L)]
                idx = lanes + (base + u * L)
                cond = v > m
                m = jnp.where(cond, v, m)
                mi = jnp.where(cond, idx, mi)
            return m, mi

        m0 = jnp.full((L,), -jnp.inf, jnp.float32)
        i0 = jnp.zeros((L,), jnp.int32)
        m, mi = lax.fori_loop(0, NCHUNK // UNROLL, pass_max, (m0, i0))

        last = jnp.full((L,), L - 1, jnp.int32)
        lp_v[...] = plsc.cummax(m)
        gmax_b = plsc.load_gather(lp_v, [last])
        cand = jnp.where(m == gmax_b, mi, jnp.int32(2**31 - 1))
        mode_v[...] = plsc.cummax(-cand)
        gmi_b = -plsc.load_gather(mode_v, [last])

        def pass_sumexp(k, s):
            base = k * (L * UNROLL)
            for u in range(UNROLL):
                v = row_v[pl.ds(base + u * L, L)]
                s = s + jnp.exp(v - gmax_b)
            return s

        s = lax.fori_loop(
            0, NCHUNK // UNROLL, pass_sumexp, jnp.zeros((L,), jnp.float32)
        )
        lp_v[...] = plsc.cumsum(s)
        ssum_b = plsc.load_gather(lp_v, [last])

        r_b = jnp.full((L,), r, jnp.int32)
        a_vec = plsc.load_gather(act_v, [r_b])
        g_vec = plsc.load_gather(row_v, [a_vec])
        lp_vec = g_vec - gmax_b - _ln(ssum_b)

        sel = lanes == j
        lp_acc = jnp.where(sel, lp_vec, lp_acc)
        mode_acc = jnp.where(sel, gmi_b, mode_acc)

    lp_v[...] = lp_acc
    mode_v[...] = mode_acc
    pltpu.sync_copy(lp_v, lp_hbm.at[wid])
    pltpu.sync_copy(mode_v, mode_hbm.at[wid])


def kernel(logits, actions):
    acts = actions.reshape(-1).astype(jnp.int32)
    lp_w, mode_w = _sc_kern(logits, acts)
    lp = lp_w[:, :RPW].reshape(B, 1)
    mode = mode_w[:, :RPW].reshape(B, 1)
    return lp, mode
